# baseline (speedup 1.0000x reference)
import functools

import jax
import jax.numpy as jnp
from jax import lax
from jax.experimental import pallas as pl
from jax.experimental.pallas import tpu as pltpu
from jax.experimental.pallas import tpu_sc as plsc

_B = 4
_T = 2048
_D = 4096
_E = 64
_K = 8
_EPS = 1e-6

_BT = _B * _T
_HALF = _BT // 2
_LANES = 16
_WORKERS = 32
_CHUNK = _HALF // _WORKERS
_GROUPS = _CHUNK // _LANES
_NEG_INF = float("-inf")


def _scale_body(hs_blk, scale_out):
    s = jnp.maximum(jnp.max(jnp.abs(hs_blk[...])), _EPS)
    scale_out[...] = jnp.full((1, _LANES), s, jnp.float32)


def _router_body(hs_hbm, scale_hbm, w_hbm, topi_hbm, x_v, w_v, ti_v, s_v):
    wid = lax.axis_index("s") * 2 + lax.axis_index("c")
    pltpu.sync_copy(hs_hbm.at[pl.ds(wid * _CHUNK * _E, _CHUNK * _E)], x_v)
    pltpu.sync_copy(scale_hbm, s_v)
    inv_s = 1.0 / s_v[...]
    lanes = lax.iota(jnp.int32, _LANES)
    zeros16 = jnp.zeros((_LANES,), jnp.float32)

    def zero(z, carry):
        for j in range(_CHUNK // _LANES):
            w_v[z, pl.ds(j * _LANES, _LANES)] = zeros16
        return carry

    lax.fori_loop(0, _E, zero, 0)

    def group(g, carry):
        toff = g * _LANES
        tok = toff + lanes
        ms = []
        as_ = []
        for k in range(_K):
            m = jnp.full((_LANES,), _NEG_INF, jnp.float32)
            a = jnp.zeros((_LANES,), jnp.int32)
            for e in range(_E):
                xe = x_v[pl.ds(e * _CHUNK + toff, _LANES)]
                gt = xe > m
                m = jnp.where(gt, xe, m)
                a = jnp.where(gt, jnp.full((_LANES,), e, jnp.int32), a)
            if k < _K - 1:
                plsc.store_scatter(
                    x_v, [a * _CHUNK + tok], jnp.full((_LANES,), _NEG_INF, jnp.float32)
                )
            ms.append(m)
            as_.append(a)
        es = [jnp.exp((mk - ms[0]) * inv_s) for mk in ms]
        denom = es[0]
        for ek in es[1:]:
            denom = denom + ek
        r = 1.0 / denom
        for k in range(_K):
            plsc.store_scatter(w_v, [as_[k], tok], es[k] * r)
            plsc.store_scatter(ti_v, [tok * _K + k], as_[k])
        return carry

    lax.fori_loop(0, _GROUPS, group, 0)
    pltpu.sync_copy(w_v, w_hbm.at[pl.ds(wid * _E, _E)])
    pltpu.sync_copy(ti_v, topi_hbm.at[pl.ds(wid * _CHUNK * _K, _CHUNK * _K)])


_router = functools.partial(
    pl.kernel,
    mesh=plsc.VectorSubcoreMesh(core_axis_name="c", subcore_axis_name="s"),
    compiler_params=pltpu.CompilerParams(needs_layout_passes=False),
    out_type=[
        jax.ShapeDtypeStruct((_WORKERS * _E, _CHUNK), jnp.float32),
        jax.ShapeDtypeStruct((_HALF * _K,), jnp.int32),
    ],
    scratch_types=[
        pltpu.VMEM((_E * _CHUNK,), jnp.float32),
        pltpu.VMEM((_E, _CHUNK), jnp.float32),
        pltpu.VMEM((_K * _CHUNK,), jnp.int32),
        pltpu.VMEM((_LANES,), jnp.float32),
    ],
)(_router_body)

_WPS = 4
_MIX_STEPS = _WORKERS // _WPS


def _mix_body(w_blk, limes, out_ref):
    for c in range(_WPS):
        out_ref[pl.ds(c * _CHUNK, _CHUNK), :] = jax.lax.dot_general(
            w_blk[pl.ds(c * _E, _E), :],
            limes[...],
            (((0,), (0,)), ((), ())),
            preferred_element_type=jnp.float32,
        )


def _mix_body1(prev_ref, w_blk, limes, out_ref):
    del prev_ref
    _mix_body(w_blk, limes, out_ref)


def _mix_half0(w2d, limes):
    return pl.pallas_call(
        _mix_body,
        grid=(_MIX_STEPS,),
        in_specs=[
            pl.BlockSpec((_WPS * _E, _CHUNK), lambda i: (i, 0)),
            pl.BlockSpec((_E, _D), lambda i: (0, 0)),
        ],
        out_specs=pl.BlockSpec((_WPS * _CHUNK, _D), lambda i: (i, 0)),
        out_shape=jax.ShapeDtypeStruct((_BT, _D), jnp.float32),
    )(w2d, limes)


def _mix_half1(prev_out, w2d, limes):
    return pl.pallas_call(
        _mix_body1,
        grid=(_MIX_STEPS,),
        in_specs=[
            pl.BlockSpec(memory_space=pltpu.MemorySpace.HBM),
            pl.BlockSpec((_WPS * _E, _CHUNK), lambda i: (i, 0)),
            pl.BlockSpec((_E, _D), lambda i: (0, 0)),
        ],
        out_specs=pl.BlockSpec((_WPS * _CHUNK, _D), lambda i: (i + _MIX_STEPS, 0)),
        out_shape=jax.ShapeDtypeStruct((_BT, _D), jnp.float32),
        input_output_aliases={0: 0},
    )(prev_out, w2d, limes)


def kernel(H, LiMEs):
    H2 = H.reshape(_BT, _D)
    staged = jnp.swapaxes(
        H2[:, :_E].reshape(2, _WORKERS, _CHUNK, _E), 2, 3
    )
    scale = pl.pallas_call(
        _scale_body,
        grid=(1,),
        in_specs=[pl.BlockSpec((2 * _WORKERS * _E * _CHUNK,), lambda i: (0,))],
        out_specs=pl.BlockSpec((1, _LANES), lambda i: (0, 0)),
        out_shape=jax.ShapeDtypeStruct((1, _LANES), jnp.float32),
    )(staged.reshape(-1))
    scale_flat = scale.reshape(-1)
    w0, ti0 = _router(staged[0].reshape(-1), scale_flat)
    w1, ti1 = _router(staged[1].reshape(-1), scale_flat)
    out0 = _mix_half0(w0, LiMEs)
    out = _mix_half1(out0, w1, LiMEs)
    p_mix = out.reshape(_B, _T, _D)
    topk_idx = jnp.concatenate([ti0, ti1]).reshape(_B, _T, _K)
    return p_mix, topk_idx


# speedup vs baseline: 1.1403x; 1.1403x over previous
import functools

import jax
import jax.numpy as jnp
from jax import lax
from jax.experimental import pallas as pl
from jax.experimental.pallas import tpu as pltpu
from jax.experimental.pallas import tpu_sc as plsc

_B = 4
_T = 2048
_D = 4096
_E = 64
_K = 8
_EPS = 1e-6

_BT = _B * _T
_LANES = 16
_WORKERS = 32
_CHUNK = _BT // _WORKERS
_GROUPS = _CHUNK // _LANES
_NEG_INF = float("-inf")


def _scale_body(hs_blk, scale_out):
    s = jnp.maximum(jnp.max(jnp.abs(hs_blk[...])), _EPS)
    scale_out[...] = jnp.full((1, _LANES), s, jnp.float32)


def _router_body(hs_hbm, scale_hbm, w_hbm, topi_hbm, x_v, w_v, ti_v, s_v):
    wid = lax.axis_index("s") * 2 + lax.axis_index("c")
    pltpu.sync_copy(hs_hbm.at[pl.ds(wid * _CHUNK * _E, _CHUNK * _E)], x_v)
    pltpu.sync_copy(scale_hbm, s_v)
    inv_s = 1.0 / s_v[...]
    lanes = lax.iota(jnp.int32, _LANES)
    zeros16 = jnp.zeros((_LANES,), jnp.float32)

    def zero(z, carry):
        for j in range(_LANES):
            w_v[z, pl.ds(j * _LANES, _LANES)] = zeros16
        return carry

    lax.fori_loop(0, _E, zero, 0)

    def group(g, carry):
        toff = g * _LANES
        tok = toff + lanes
        ms = []
        as_ = []
        for k in range(_K):
            m = jnp.full((_LANES,), _NEG_INF, jnp.float32)
            a = jnp.zeros((_LANES,), jnp.int32)
            for e in range(_E):
                xe = x_v[pl.ds(e * _CHUNK + toff, _LANES)]
                gt = xe > m
                m = jnp.where(gt, xe, m)
                a = jnp.where(gt, jnp.full((_LANES,), e, jnp.int32), a)
            if k < _K - 1:
                plsc.store_scatter(
                    x_v, [a * _CHUNK + tok], jnp.full((_LANES,), _NEG_INF, jnp.float32)
                )
            ms.append(m)
            as_.append(a)
        es = [jnp.exp((mk - ms[0]) * inv_s) for mk in ms]
        denom = es[0]
        for ek in es[1:]:
            denom = denom + ek
        r = 1.0 / denom
        for k in range(_K):
            plsc.store_scatter(w_v, [as_[k], tok], es[k] * r)
            plsc.store_scatter(ti_v, [tok * _K + k], as_[k])
        return carry

    lax.fori_loop(0, _GROUPS, group, 0)
    pltpu.sync_copy(w_v, w_hbm.at[pl.ds(wid * _E, _E)])
    pltpu.sync_copy(ti_v, topi_hbm.at[pl.ds(wid * _CHUNK * _K, _CHUNK * _K)])


_router = functools.partial(
    pl.kernel,
    mesh=plsc.VectorSubcoreMesh(core_axis_name="c", subcore_axis_name="s"),
    compiler_params=pltpu.CompilerParams(needs_layout_passes=False),
    out_type=[
        jax.ShapeDtypeStruct((_WORKERS * _E, _CHUNK), jnp.float32),
        jax.ShapeDtypeStruct((_BT * _K,), jnp.int32),
    ],
    scratch_types=[
        pltpu.VMEM((_E * _CHUNK,), jnp.float32),
        pltpu.VMEM((_E, _CHUNK), jnp.float32),
        pltpu.VMEM((_K * _CHUNK,), jnp.int32),
        pltpu.VMEM((_LANES,), jnp.float32),
    ],
)(_router_body)


def _mix_body(w_blk, limes, out_ref):
    for c in range(2):
        out_ref[pl.ds(c * _CHUNK, _CHUNK), :] = jax.lax.dot_general(
            w_blk[pl.ds(c * _E, _E), :],
            limes[...],
            (((0,), (0,)), ((), ())),
            preferred_element_type=jnp.float32,
        )


def kernel(H, LiMEs):
    H2 = H.reshape(_BT, _D)
    hs_prep = (
        jnp.swapaxes(H2[:, :_E].reshape(_WORKERS, _CHUNK, _E), 1, 2).reshape(-1)
    )
    scale = pl.pallas_call(
        _scale_body,
        grid=(1,),
        in_specs=[pl.BlockSpec((_WORKERS * _E * _CHUNK,), lambda i: (0,))],
        out_specs=pl.BlockSpec((1, _LANES), lambda i: (0, 0)),
        out_shape=jax.ShapeDtypeStruct((1, _LANES), jnp.float32),
    )(hs_prep)
    w_flat, topi_flat = _router(hs_prep, scale.reshape(-1))
    out = pl.pallas_call(
        _mix_body,
        grid=(_WORKERS // 2,),
        in_specs=[
            pl.BlockSpec((2 * _E, _CHUNK), lambda i: (i, 0)),
            pl.BlockSpec((_E, _D), lambda i: (0, 0)),
        ],
        out_specs=pl.BlockSpec((2 * _CHUNK, _D), lambda i: (i, 0)),
        out_shape=jax.ShapeDtypeStruct((_BT, _D), jnp.float32),
        compiler_params=pltpu.CompilerParams(fuse_transposed_lhs_in_matmul=True),
    )(w_flat, LiMEs)
    p_mix = out.reshape(_B, _T, _D)
    topk_idx = topi_flat.reshape(_B, _T, _K)
    return p_mix, topk_idx
